# Initial kernel scaffold; baseline (speedup 1.0000x reference)
import functools

import jax
import jax.numpy as jnp
from jax import lax
from jax.experimental import pallas as pl
from jax.experimental.pallas import tpu as pltpu
from jax.experimental.pallas import tpu_sc as plsc

_N_GENE = 2500
_N_TRAIN = 10000
_E = 32000
_SRC_DIM = 2500
_DST_DIM = 2675

_CPAD = 2560
_NCHUNK = _CPAD // 128
_KTRAIN = 2688
_DEG_COL = 2500

_EPAD = 32768
_BE = 128
_NB = _EPAD // (16 * _BE)
_EROWS = _EPAD // _BE
_ZROW = _NCHUNK * _N_GENE
_GROWS = _ZROW + 16
_STRIPE = _N_TRAIN // 16

_BM = 1000
_BN = 1280


def _sc_segment_sum(gflat, gidx, edst):
    mesh = plsc.VectorSubcoreMesh(core_axis_name="c", subcore_axis_name="s")

    @functools.partial(
        pl.kernel,
        mesh=mesh,
        out_type=jax.ShapeDtypeStruct((_N_TRAIN, _CPAD), jnp.float32),
        scratch_types=[
            pltpu.VMEM((_BE,), jnp.int32),
            pltpu.VMEM((_BE, 128), jnp.float32),
            pltpu.VMEM((_NB, _BE), jnp.int32),
            pltpu.VMEM((125, 128), jnp.float32),
            pltpu.VMEM_SHARED((_N_TRAIN, 128), jnp.float32),
            pltpu.SemaphoreType.DMA,
        ],
    )
    def k(gflat_hbm, gidx_hbm, edst_hbm, out_hbm,
          idx_v, rows_v, dst_v, zeros_v, acc_sh, sem):
        c = lax.axis_index("c")
        s = lax.axis_index("s")

        pltpu.sync_copy(edst_hbm.at[pl.ds(s * _NB, _NB)], dst_v)

        zero16 = jnp.zeros((16,), jnp.float32)

        @pl.loop(0, 125)
        def _(i):
            @pl.loop(0, 128, step=16)
            def _(j):
                zeros_v[i, pl.ds(j, 16)] = zero16

        @pl.loop(0, _STRIPE // 125)
        def _(z):
            pltpu.sync_copy(zeros_v, acc_sh.at[pl.ds(s * _STRIPE + z * 125, 125)])

        plsc.subcore_barrier()

        @pl.loop(0, _NCHUNK // 2)
        def _(cc):
            chunk = c * (_NCHUNK // 2) + cc

            @pl.loop(0, _NB)
            def _(b):
                pltpu.sync_copy(
                    gidx_hbm.at[chunk * _EROWS + s * _NB + b], idx_v)
                pltpu.async_copy(gflat_hbm.at[idx_v], rows_v, sem).wait()
                pltpu.sync_copy(rows_v, acc_sh.at[dst_v.at[b]], add=True)

            plsc.subcore_barrier()

            @pl.loop(0, _STRIPE // 125)
            def _(z):
                r0 = s * _STRIPE + z * 125
                pltpu.sync_copy(
                    acc_sh.at[pl.ds(r0, 125)],
                    out_hbm.at[pl.ds(r0, 125), pl.ds(chunk * 128, 128)])
                pltpu.sync_copy(zeros_v, acc_sh.at[pl.ds(r0, 125)])

            plsc.subcore_barrier()

    return k(gflat, gidx, edst)


def _mm_bias_body(x_ref, w_ref, b_ref, o_ref):
    acc = jnp.dot(x_ref[...], w_ref[...], preferred_element_type=jnp.float32)
    acc += b_ref[...].astype(jnp.float32)
    o_ref[...] = acc.astype(jnp.bfloat16)


def _mm_bias(x, w, b):
    m, kdim = x.shape
    n = w.shape[1]
    grid = (n // _BN, m // _BM)
    return pl.pallas_call(
        _mm_bias_body,
        grid=grid,
        in_specs=[
            pl.BlockSpec((_BM, kdim), lambda ni, mi: (mi, 0)),
            pl.BlockSpec((kdim, _BN), lambda ni, mi: (0, ni)),
            pl.BlockSpec((1, _BN), lambda ni, mi: (0, ni)),
        ],
        out_specs=pl.BlockSpec((_BM, _BN), lambda ni, mi: (mi, ni)),
        out_shape=jax.ShapeDtypeStruct((m, n), jnp.bfloat16),
        compiler_params=pltpu.CompilerParams(
            dimension_semantics=("arbitrary", "arbitrary")),
    )(x, w, b)


def _scaled_agg(ag_ref):
    deg = ag_ref[:, _DEG_COL].astype(jnp.float32)
    r = 1.0 / jnp.maximum(deg, 1.0)
    a = ag_ref[...].astype(jnp.float32) * r[:, None]
    return a.astype(jnp.bfloat16)


def _combine1_body(s_ref, ag_ref, wn_ref, o_ref):
    acc = s_ref[...].astype(jnp.float32)
    acc += jnp.dot(_scaled_agg(ag_ref), wn_ref[...],
                   preferred_element_type=jnp.float32)
    o_ref[...] = jnp.maximum(acc, 0.0).astype(jnp.bfloat16)


def _combine1(s1, agg, wn):
    m, n = s1.shape
    grid = (n // _BN, m // _BM)
    return pl.pallas_call(
        _combine1_body,
        grid=grid,
        in_specs=[
            pl.BlockSpec((_BM, _BN), lambda ni, mi: (mi, ni)),
            pl.BlockSpec((_BM, _CPAD), lambda ni, mi: (mi, 0)),
            pl.BlockSpec((_CPAD, _BN), lambda ni, mi: (0, ni)),
        ],
        out_specs=pl.BlockSpec((_BM, _BN), lambda ni, mi: (mi, ni)),
        out_shape=jax.ShapeDtypeStruct((m, n), jnp.bfloat16),
        compiler_params=pltpu.CompilerParams(
            dimension_semantics=("arbitrary", "arbitrary")),
    )(s1, agg, wn)


def _layer2_body(h_ref, ag_ref, ws_ref, wn_ref, b_ref, o_ref):
    acc = jnp.dot(h_ref[...], ws_ref[...], preferred_element_type=jnp.float32)
    acc += jnp.dot(_scaled_agg(ag_ref), wn_ref[...],
                   preferred_element_type=jnp.float32)
    acc += b_ref[...].astype(jnp.float32)
    o_ref[...] = jnp.maximum(acc, 0.0).astype(jnp.bfloat16)


def _layer2(h, agg, ws, wn, b):
    m, kdim = h.shape
    n = ws.shape[1]
    grid = (n // _BN, m // _BM)
    return pl.pallas_call(
        _layer2_body,
        grid=grid,
        in_specs=[
            pl.BlockSpec((_BM, kdim), lambda ni, mi: (mi, 0)),
            pl.BlockSpec((_BM, _CPAD), lambda ni, mi: (mi, 0)),
            pl.BlockSpec((kdim, _BN), lambda ni, mi: (0, ni)),
            pl.BlockSpec((_CPAD, _BN), lambda ni, mi: (0, ni)),
            pl.BlockSpec((1, _BN), lambda ni, mi: (0, ni)),
        ],
        out_specs=pl.BlockSpec((_BM, _BN), lambda ni, mi: (mi, ni)),
        out_shape=jax.ShapeDtypeStruct((m, n), jnp.bfloat16),
        compiler_params=pltpu.CompilerParams(
            dimension_semantics=("arbitrary", "arbitrary")),
    )(h, agg, ws, wn, b)


def _head_body(h2_ref, wc1_ref, bc1_ref, wc2_ref, bc2_ref, o_ref):
    ni = pl.program_id(1)
    t = jnp.dot(h2_ref[...], wc1_ref[...], preferred_element_type=jnp.float32)
    t = jnp.maximum(t + bc1_ref[...].astype(jnp.float32), 0.0)
    part = jnp.dot(t.astype(jnp.bfloat16), wc2_ref[...],
                   preferred_element_type=jnp.float32)

    @pl.when(ni == 0)
    def _():
        o_ref[...] = part + bc2_ref[...].astype(jnp.float32)

    @pl.when(ni > 0)
    def _():
        o_ref[...] += part


def _head(h2, wc1, bc1, wc2, bc2):
    m, kdim = h2.shape
    ncls = wc2.shape[1]
    grid = (m // _BM, kdim // _BN)
    return pl.pallas_call(
        _head_body,
        grid=grid,
        in_specs=[
            pl.BlockSpec((_BM, kdim), lambda mi, ni: (mi, 0)),
            pl.BlockSpec((kdim, _BN), lambda mi, ni: (0, ni)),
            pl.BlockSpec((1, _BN), lambda mi, ni: (0, ni)),
            pl.BlockSpec((_BN, ncls), lambda mi, ni: (ni, 0)),
            pl.BlockSpec((1, ncls), lambda mi, ni: (0, 0)),
        ],
        out_specs=pl.BlockSpec((_BM, ncls), lambda mi, ni: (mi, 0)),
        out_shape=jax.ShapeDtypeStruct((m, ncls), jnp.float32),
        compiler_params=pltpu.CompilerParams(
            dimension_semantics=("arbitrary", "arbitrary")),
    )(h2, wc1, bc1, wc2, bc2)


def _pad2(x, rows, cols):
    return jnp.pad(x, ((0, rows - x.shape[0]), (0, cols - x.shape[1])))


def kernel(gene_features, train_features, edge_src, edge_dst,
           W_self1, W_neigh1, b1, W_self2, W_neigh2, b2,
           Wc1, bc1, Wc2, bc2):
    f32, bf16 = jnp.float32, jnp.bfloat16

    gp = jnp.concatenate(
        [gene_features,
         jnp.ones((_N_GENE, 1), f32),
         jnp.zeros((_N_GENE, _CPAD - _SRC_DIM - 1), f32)], axis=1)
    gflat = gp.reshape(_N_GENE, _NCHUNK, 128).transpose(1, 0, 2)
    gflat = gflat.reshape(_ZROW, 128)
    gflat = jnp.pad(gflat, ((0, _GROWS - _ZROW), (0, 0)))

    valid = jnp.arange(_EPAD, dtype=jnp.int32) < _E
    src_pad = jnp.pad(edge_src, (0, _EPAD - _E))
    base = jnp.arange(_NCHUNK, dtype=jnp.int32)[:, None] * _N_GENE
    gidx = jnp.where(valid[None, :], src_pad[None, :] + base, _ZROW)
    gidx = gidx.astype(jnp.int32).reshape(_NCHUNK * _EROWS, _BE)
    edst = jnp.pad(edge_dst, (0, _EPAD - _E)).reshape(_EROWS, _BE)

    tfp = _pad2(train_features, _N_TRAIN, _KTRAIN).astype(bf16)
    w1s = _pad2(W_self1, _KTRAIN, _CPAD).astype(bf16)
    w1n = _pad2(W_neigh1, _CPAD, _CPAD).astype(bf16)
    w2s = _pad2(W_self2, _CPAD, _CPAD).astype(bf16)
    w2n = _pad2(W_neigh2, _CPAD, _CPAD).astype(bf16)
    wc1 = _pad2(Wc1, _CPAD, _CPAD).astype(bf16)
    wc2 = _pad2(Wc2, _CPAD, Wc2.shape[1]).astype(bf16)
    b1p = jnp.pad(b1, (0, _CPAD - b1.shape[0])).reshape(1, _CPAD)
    b2p = jnp.pad(b2, (0, _CPAD - b2.shape[0])).reshape(1, _CPAD)
    bc1p = jnp.pad(bc1, (0, _CPAD - bc1.shape[0])).reshape(1, _CPAD)
    bc2p = bc2.reshape(1, -1)

    aggraw = _sc_segment_sum(gflat, gidx, edst)
    s1 = _mm_bias(tfp, w1s, b1p)
    aggb = aggraw.astype(bf16)
    h = _combine1(s1, aggb, w1n)
    h2 = _layer2(h, aggb, w2s, w2n, b2p)
    out = _head(h2, wc1, bc1p, wc2, bc2p)
    return out


# speedup vs baseline: 1.3230x; 1.3230x over previous
import functools

import jax
import jax.numpy as jnp
from jax import lax
from jax.experimental import pallas as pl
from jax.experimental.pallas import tpu as pltpu
from jax.experimental.pallas import tpu_sc as plsc

_N_GENE = 2500
_N_TRAIN = 10000
_E = 32000
_SRC_DIM = 2500
_DST_DIM = 2675

_CPAD = 2560
_NCHUNK = _CPAD // 128
_KTRAIN = 2688
_DEG_COL = 2500

_EPAD = 32768
_BE = 128
_NB = _EPAD // (16 * _BE)
_EROWS = _EPAD // _BE
_ZROW = _NCHUNK * _N_GENE
_GROWS = _ZROW + 16
_MPAD = 10240
_STRIPE = _MPAD // 16

_BM = 512
_BN = 1280


def _sc_segment_sum(gflat, gidx, edst):
    mesh = plsc.VectorSubcoreMesh(core_axis_name="c", subcore_axis_name="s")

    @functools.partial(
        pl.kernel,
        mesh=mesh,
        out_type=jax.ShapeDtypeStruct((_MPAD, _CPAD), jnp.float32),
        scratch_types=[
            pltpu.VMEM((8, _BE), jnp.int32),
            pltpu.VMEM((_BE, 128), jnp.float32),
            pltpu.VMEM((_NB, _BE), jnp.int32),
            pltpu.VMEM((128, 128), jnp.float32),
            pltpu.VMEM_SHARED((_MPAD, 128), jnp.float32),
            pltpu.SemaphoreType.DMA,
        ],
    )
    def k(gflat_hbm, gidx_hbm, edst_hbm, out_hbm,
          idx_v, rows_v, dst_v, zeros_v, acc_sh, sem):
        c = lax.axis_index("c")
        s = lax.axis_index("s")

        pltpu.sync_copy(edst_hbm.at[pl.ds(s * _NB, _NB)], dst_v)

        zero16 = jnp.zeros((16,), jnp.float32)

        @pl.loop(0, 128)
        def _(i):
            @pl.loop(0, 128, step=16)
            def _(j):
                zeros_v[i, pl.ds(j, 16)] = zero16

        @pl.loop(0, _STRIPE // 128)
        def _(z):
            pltpu.sync_copy(zeros_v, acc_sh.at[pl.ds(s * _STRIPE + z * 128, 128)])

        plsc.subcore_barrier()

        @pl.loop(0, _NCHUNK // 2)
        def _(cc):
            chunk = c * (_NCHUNK // 2) + cc

            @pl.loop(0, _NB // 8)
            def _(half):
                pltpu.sync_copy(
                    gidx_hbm.at[pl.ds(chunk * _EROWS + s * _NB + half * 8, 8)],
                    idx_v)

                @pl.loop(0, 8)
                def _(b):
                    pltpu.async_copy(gflat_hbm.at[idx_v.at[b]], rows_v,
                                     sem).wait()
                    pltpu.sync_copy(rows_v, acc_sh.at[dst_v.at[half * 8 + b]],
                                    add=True)

            plsc.subcore_barrier()

            @pl.loop(0, _STRIPE // 128)
            def _(z):
                r0 = s * _STRIPE + z * 128
                pltpu.sync_copy(
                    acc_sh.at[pl.ds(r0, 128)],
                    out_hbm.at[pl.ds(r0, 128), pl.ds(chunk * 128, 128)])
                pltpu.sync_copy(zeros_v, acc_sh.at[pl.ds(r0, 128)])

            plsc.subcore_barrier()

    return k(gflat, gidx, edst)


def _mm_bias_body(x_ref, w_ref, b_ref, o_ref):
    acc = jnp.dot(x_ref[...], w_ref[...], preferred_element_type=jnp.float32)
    acc += b_ref[...].astype(jnp.float32)
    o_ref[...] = acc.astype(jnp.bfloat16)


def _mm_bias(x, w, b):
    m, kdim = x.shape
    n = w.shape[1]
    grid = (n // _BN, m // _BM)
    return pl.pallas_call(
        _mm_bias_body,
        grid=grid,
        in_specs=[
            pl.BlockSpec((_BM, kdim), lambda ni, mi: (mi, 0)),
            pl.BlockSpec((kdim, _BN), lambda ni, mi: (0, ni)),
            pl.BlockSpec((1, _BN), lambda ni, mi: (0, ni)),
        ],
        out_specs=pl.BlockSpec((_BM, _BN), lambda ni, mi: (mi, ni)),
        out_shape=jax.ShapeDtypeStruct((m, n), jnp.bfloat16),
        compiler_params=pltpu.CompilerParams(
            dimension_semantics=("arbitrary", "arbitrary")),
    )(x, w, b)


def _scaled_agg(ag_ref):
    deg = ag_ref[:, _DEG_COL].astype(jnp.float32)
    r = 1.0 / jnp.maximum(deg, 1.0)
    a = ag_ref[...].astype(jnp.float32) * r[:, None]
    return a.astype(jnp.bfloat16)


def _combine1_body(s_ref, ag_ref, wn_ref, o_ref):
    acc = s_ref[...].astype(jnp.float32)
    acc += jnp.dot(_scaled_agg(ag_ref), wn_ref[...],
                   preferred_element_type=jnp.float32)
    o_ref[...] = jnp.maximum(acc, 0.0).astype(jnp.bfloat16)


def _combine1(s1, agg, wn):
    m, n = s1.shape
    grid = (n // _BN, m // _BM)
    return pl.pallas_call(
        _combine1_body,
        grid=grid,
        in_specs=[
            pl.BlockSpec((_BM, _BN), lambda ni, mi: (mi, ni)),
            pl.BlockSpec((_BM, _CPAD), lambda ni, mi: (mi, 0)),
            pl.BlockSpec((_CPAD, _BN), lambda ni, mi: (0, ni)),
        ],
        out_specs=pl.BlockSpec((_BM, _BN), lambda ni, mi: (mi, ni)),
        out_shape=jax.ShapeDtypeStruct((m, n), jnp.bfloat16),
        compiler_params=pltpu.CompilerParams(
            dimension_semantics=("arbitrary", "arbitrary")),
    )(s1, agg, wn)


def _layer2_body(h_ref, ag_ref, ws_ref, wn_ref, b_ref, o_ref):
    acc = jnp.dot(h_ref[...], ws_ref[...], preferred_element_type=jnp.float32)
    acc += jnp.dot(_scaled_agg(ag_ref), wn_ref[...],
                   preferred_element_type=jnp.float32)
    acc += b_ref[...].astype(jnp.float32)
    o_ref[...] = jnp.maximum(acc, 0.0).astype(jnp.bfloat16)


def _layer2(h, agg, ws, wn, b):
    m, kdim = h.shape
    n = ws.shape[1]
    grid = (n // _BN, m // _BM)
    return pl.pallas_call(
        _layer2_body,
        grid=grid,
        in_specs=[
            pl.BlockSpec((_BM, kdim), lambda ni, mi: (mi, 0)),
            pl.BlockSpec((_BM, _CPAD), lambda ni, mi: (mi, 0)),
            pl.BlockSpec((kdim, _BN), lambda ni, mi: (0, ni)),
            pl.BlockSpec((_CPAD, _BN), lambda ni, mi: (0, ni)),
            pl.BlockSpec((1, _BN), lambda ni, mi: (0, ni)),
        ],
        out_specs=pl.BlockSpec((_BM, _BN), lambda ni, mi: (mi, ni)),
        out_shape=jax.ShapeDtypeStruct((m, n), jnp.bfloat16),
        compiler_params=pltpu.CompilerParams(
            dimension_semantics=("arbitrary", "arbitrary")),
    )(h, agg, ws, wn, b)


def _head_body(h2_ref, wc1_ref, bc1_ref, wc2_ref, bc2_ref, o_ref):
    ni = pl.program_id(1)
    t = jnp.dot(h2_ref[...], wc1_ref[...], preferred_element_type=jnp.float32)
    t = jnp.maximum(t + bc1_ref[...].astype(jnp.float32), 0.0)
    part = jnp.dot(t.astype(jnp.bfloat16), wc2_ref[...],
                   preferred_element_type=jnp.float32)

    @pl.when(ni == 0)
    def _():
        o_ref[...] = part + bc2_ref[...].astype(jnp.float32)

    @pl.when(ni > 0)
    def _():
        o_ref[...] += part


def _head(h2, wc1, bc1, wc2, bc2):
    m, kdim = h2.shape
    ncls = wc2.shape[1]
    grid = (m // _BM, kdim // _BN)
    return pl.pallas_call(
        _head_body,
        grid=grid,
        in_specs=[
            pl.BlockSpec((_BM, kdim), lambda mi, ni: (mi, 0)),
            pl.BlockSpec((kdim, _BN), lambda mi, ni: (0, ni)),
            pl.BlockSpec((1, _BN), lambda mi, ni: (0, ni)),
            pl.BlockSpec((_BN, ncls), lambda mi, ni: (ni, 0)),
            pl.BlockSpec((1, ncls), lambda mi, ni: (0, 0)),
        ],
        out_specs=pl.BlockSpec((_BM, ncls), lambda mi, ni: (mi, 0)),
        out_shape=jax.ShapeDtypeStruct((m, ncls), jnp.float32),
        compiler_params=pltpu.CompilerParams(
            dimension_semantics=("arbitrary", "arbitrary")),
    )(h2, wc1, bc1, wc2, bc2)


def _pad2(x, rows, cols):
    return jnp.pad(x, ((0, rows - x.shape[0]), (0, cols - x.shape[1])))


def kernel(gene_features, train_features, edge_src, edge_dst,
           W_self1, W_neigh1, b1, W_self2, W_neigh2, b2,
           Wc1, bc1, Wc2, bc2):
    f32, bf16 = jnp.float32, jnp.bfloat16

    gp = jnp.concatenate(
        [gene_features,
         jnp.ones((_N_GENE, 1), f32),
         jnp.zeros((_N_GENE, _CPAD - _SRC_DIM - 1), f32)], axis=1)
    gflat = gp.reshape(_N_GENE, _NCHUNK, 128).transpose(1, 0, 2)
    gflat = gflat.reshape(_ZROW, 128)
    gflat = jnp.pad(gflat, ((0, _GROWS - _ZROW), (0, 0)))

    valid = jnp.arange(_EPAD, dtype=jnp.int32) < _E
    src_pad = jnp.pad(edge_src, (0, _EPAD - _E))
    base = jnp.arange(_NCHUNK, dtype=jnp.int32)[:, None] * _N_GENE
    gidx = jnp.where(valid[None, :], src_pad[None, :] + base, _ZROW)
    gidx = gidx.astype(jnp.int32).reshape(_NCHUNK * _EROWS, _BE)
    edst = jnp.pad(edge_dst, (0, _EPAD - _E)).reshape(_EROWS, _BE)

    tfp = _pad2(train_features, _MPAD, _KTRAIN).astype(bf16)
    w1s = _pad2(W_self1, _KTRAIN, _CPAD).astype(bf16)
    w1n = _pad2(W_neigh1, _CPAD, _CPAD).astype(bf16)
    w2s = _pad2(W_self2, _CPAD, _CPAD).astype(bf16)
    w2n = _pad2(W_neigh2, _CPAD, _CPAD).astype(bf16)
    wc1 = _pad2(Wc1, _CPAD, _CPAD).astype(bf16)
    wc2 = _pad2(Wc2, _CPAD, Wc2.shape[1]).astype(bf16)
    b1p = jnp.pad(b1, (0, _CPAD - b1.shape[0])).reshape(1, _CPAD)
    b2p = jnp.pad(b2, (0, _CPAD - b2.shape[0])).reshape(1, _CPAD)
    bc1p = jnp.pad(bc1, (0, _CPAD - bc1.shape[0])).reshape(1, _CPAD)
    bc2p = bc2.reshape(1, -1)

    aggraw = _sc_segment_sum(gflat, gidx, edst)
    s1 = _mm_bias(tfp, w1s, b1p)
    aggb = aggraw.astype(bf16)
    h = _combine1(s1, aggb, w1n)
    h2 = _layer2(h, aggb, w2s, w2n, b2p)
    out = _head(h2, wc1, bc1p, wc2, bc2p)
    return out[:_N_TRAIN]


# speedup vs baseline: 1.5743x; 1.1900x over previous
import functools

import jax
import jax.numpy as jnp
from jax import lax
from jax.experimental import pallas as pl
from jax.experimental.pallas import tpu as pltpu
from jax.experimental.pallas import tpu_sc as plsc

_N_GENE = 2500
_N_TRAIN = 10000
_E = 32000
_SRC_DIM = 2500
_DST_DIM = 2675

_CPAD = 2560
_NCHUNK = _CPAD // 128
_KTRAIN = 2688
_DEG_COL = 2500

_EPAD = 32768
_BE = 128
_NB = _EPAD // (16 * _BE)
_EROWS = _EPAD // _BE
_ZROW = _NCHUNK * _N_GENE
_GROWS = _ZROW + 16
_MPAD = 10240
_STRIPE = _MPAD // 16

_BM = 512
_BN = 1280


def _sc_segment_sum(gflat, esrc, edst):
    mesh = plsc.VectorSubcoreMesh(core_axis_name="c", subcore_axis_name="s")

    @functools.partial(
        pl.kernel,
        mesh=mesh,
        out_type=jax.ShapeDtypeStruct((_MPAD, _CPAD), jnp.float32),
        scratch_types=[
            pltpu.VMEM((_NB, _BE), jnp.int32),
            pltpu.VMEM((_NB, _BE), jnp.int32),
            pltpu.VMEM((_NB, _BE), jnp.int32),
            pltpu.VMEM((_BE, 128), jnp.float32),
            pltpu.VMEM((_BE, 128), jnp.float32),
            pltpu.VMEM((64, 128), jnp.float32),
            pltpu.VMEM_SHARED((_MPAD, 128), jnp.float32),
            pltpu.SemaphoreType.DMA,
            pltpu.SemaphoreType.DMA,
            pltpu.SemaphoreType.DMA,
            pltpu.SemaphoreType.DMA,
        ],
    )
    def k(gflat_hbm, esrc_hbm, edst_hbm, out_hbm,
          src_v, dst_v, idx_v, rows_a, rows_b, zeros_v, acc_sh,
          sga, sgb, ssa, ssb):
        c = lax.axis_index("c")
        s = lax.axis_index("s")

        pltpu.sync_copy(esrc_hbm.at[pl.ds(s * _NB, _NB)], src_v)
        pltpu.sync_copy(edst_hbm.at[pl.ds(s * _NB, _NB)], dst_v)

        zero16 = jnp.zeros((16,), jnp.float32)

        @pl.loop(0, 64)
        def _(i):
            @pl.loop(0, 128, step=16)
            def _(j):
                zeros_v[i, pl.ds(j, 16)] = zero16

        @pl.loop(0, _STRIPE // 64)
        def _(z):
            pltpu.sync_copy(zeros_v, acc_sh.at[pl.ds(s * _STRIPE + z * 64, 64)])

        plsc.subcore_barrier()

        def gather(b, buf, sem):
            pltpu.async_copy(gflat_hbm.at[idx_v.at[b]], buf, sem)

        def wait_gather(b, buf, sem):
            pltpu.make_async_copy(gflat_hbm.at[idx_v.at[b]], buf, sem).wait()

        def scatter(b, buf, sem):
            pltpu.async_copy(buf, acc_sh.at[dst_v.at[b]], sem, add=True)

        def wait_scatter(b, buf, sem):
            pltpu.make_async_copy(buf, acc_sh.at[dst_v.at[b]], sem).wait()

        @pl.loop(0, _NCHUNK // 2)
        def _(cc):
            chunk = c * (_NCHUNK // 2) + cc

            @pl.loop(0, _NB)
            def _(i):
                @pl.loop(0, 128, step=16)
                def _(j):
                    idx_v[i, pl.ds(j, 16)] = (
                        src_v[i, pl.ds(j, 16)] * _NCHUNK + chunk)

            gather(0, rows_a, sga)

            @pl.loop(0, _NB // 2)
            def _(p):
                b0 = p * 2
                wait_gather(b0, rows_a, sga)

                @pl.when(p > 0)
                def _():
                    wait_scatter(b0 - 1, rows_b, ssb)

                gather(b0 + 1, rows_b, sgb)
                scatter(b0, rows_a, ssa)
                wait_gather(b0 + 1, rows_b, sgb)
                wait_scatter(b0, rows_a, ssa)

                @pl.when(p < _NB // 2 - 1)
                def _():
                    gather(b0 + 2, rows_a, sga)

                scatter(b0 + 1, rows_b, ssb)

            wait_scatter(_NB - 1, rows_b, ssb)
            plsc.subcore_barrier()

            @pl.loop(0, _STRIPE // 64)
            def _(z):
                r0 = s * _STRIPE + z * 64
                pltpu.sync_copy(
                    acc_sh.at[pl.ds(r0, 64)],
                    out_hbm.at[pl.ds(r0, 64), pl.ds(chunk * 128, 128)])
                pltpu.sync_copy(zeros_v, acc_sh.at[pl.ds(r0, 64)])

            plsc.subcore_barrier()

    return k(gflat, esrc, edst)


def _mm_bias_body(x_ref, w_ref, b_ref, o_ref):
    acc = jnp.dot(x_ref[...], w_ref[...], preferred_element_type=jnp.float32)
    acc += b_ref[...].astype(jnp.float32)
    o_ref[...] = acc.astype(jnp.bfloat16)


def _mm_bias(x, w, b):
    m, kdim = x.shape
    n = w.shape[1]
    grid = (n // _BN, m // _BM)
    return pl.pallas_call(
        _mm_bias_body,
        grid=grid,
        in_specs=[
            pl.BlockSpec((_BM, kdim), lambda ni, mi: (mi, 0)),
            pl.BlockSpec((kdim, _BN), lambda ni, mi: (0, ni)),
            pl.BlockSpec((1, _BN), lambda ni, mi: (0, ni)),
        ],
        out_specs=pl.BlockSpec((_BM, _BN), lambda ni, mi: (mi, ni)),
        out_shape=jax.ShapeDtypeStruct((m, n), jnp.bfloat16),
        compiler_params=pltpu.CompilerParams(
            dimension_semantics=("arbitrary", "arbitrary")),
    )(x, w, b)


def _scaled_agg(ag_ref):
    deg = ag_ref[:, _DEG_COL].astype(jnp.float32)
    r = 1.0 / jnp.maximum(deg, 1.0)
    a = ag_ref[...].astype(jnp.float32) * r[:, None]
    return a.astype(jnp.bfloat16)


def _combine1_body(s_ref, ag_ref, wn_ref, o_ref):
    acc = s_ref[...].astype(jnp.float32)
    acc += jnp.dot(_scaled_agg(ag_ref), wn_ref[...],
                   preferred_element_type=jnp.float32)
    o_ref[...] = jnp.maximum(acc, 0.0).astype(jnp.bfloat16)


def _combine1(s1, agg, wn):
    m, n = s1.shape
    grid = (n // _BN, m // _BM)
    return pl.pallas_call(
        _combine1_body,
        grid=grid,
        in_specs=[
            pl.BlockSpec((_BM, _BN), lambda ni, mi: (mi, ni)),
            pl.BlockSpec((_BM, _CPAD), lambda ni, mi: (mi, 0)),
            pl.BlockSpec((_CPAD, _BN), lambda ni, mi: (0, ni)),
        ],
        out_specs=pl.BlockSpec((_BM, _BN), lambda ni, mi: (mi, ni)),
        out_shape=jax.ShapeDtypeStruct((m, n), jnp.bfloat16),
        compiler_params=pltpu.CompilerParams(
            dimension_semantics=("arbitrary", "arbitrary")),
    )(s1, agg, wn)


def _layer2_body(h_ref, ag_ref, ws_ref, wn_ref, b_ref, o_ref):
    acc = jnp.dot(h_ref[...], ws_ref[...], preferred_element_type=jnp.float32)
    acc += jnp.dot(_scaled_agg(ag_ref), wn_ref[...],
                   preferred_element_type=jnp.float32)
    acc += b_ref[...].astype(jnp.float32)
    o_ref[...] = jnp.maximum(acc, 0.0).astype(jnp.bfloat16)


def _layer2(h, agg, ws, wn, b):
    m, kdim = h.shape
    n = ws.shape[1]
    grid = (n // _BN, m // _BM)
    return pl.pallas_call(
        _layer2_body,
        grid=grid,
        in_specs=[
            pl.BlockSpec((_BM, kdim), lambda ni, mi: (mi, 0)),
            pl.BlockSpec((_BM, _CPAD), lambda ni, mi: (mi, 0)),
            pl.BlockSpec((kdim, _BN), lambda ni, mi: (0, ni)),
            pl.BlockSpec((_CPAD, _BN), lambda ni, mi: (0, ni)),
            pl.BlockSpec((1, _BN), lambda ni, mi: (0, ni)),
        ],
        out_specs=pl.BlockSpec((_BM, _BN), lambda ni, mi: (mi, ni)),
        out_shape=jax.ShapeDtypeStruct((m, n), jnp.bfloat16),
        compiler_params=pltpu.CompilerParams(
            dimension_semantics=("arbitrary", "arbitrary")),
    )(h, agg, ws, wn, b)


def _head_body(h2_ref, wc1_ref, bc1_ref, wc2_ref, bc2_ref, o_ref):
    ni = pl.program_id(1)
    t = jnp.dot(h2_ref[...], wc1_ref[...], preferred_element_type=jnp.float32)
    t = jnp.maximum(t + bc1_ref[...].astype(jnp.float32), 0.0)
    part = jnp.dot(t.astype(jnp.bfloat16), wc2_ref[...],
                   preferred_element_type=jnp.float32)

    @pl.when(ni == 0)
    def _():
        o_ref[...] = part + bc2_ref[...].astype(jnp.float32)

    @pl.when(ni > 0)
    def _():
        o_ref[...] += part


def _head(h2, wc1, bc1, wc2, bc2):
    m, kdim = h2.shape
    ncls = wc2.shape[1]
    grid = (m // _BM, kdim // _BN)
    return pl.pallas_call(
        _head_body,
        grid=grid,
        in_specs=[
            pl.BlockSpec((_BM, kdim), lambda mi, ni: (mi, 0)),
            pl.BlockSpec((kdim, _BN), lambda mi, ni: (0, ni)),
            pl.BlockSpec((1, _BN), lambda mi, ni: (0, ni)),
            pl.BlockSpec((_BN, ncls), lambda mi, ni: (ni, 0)),
            pl.BlockSpec((1, ncls), lambda mi, ni: (0, 0)),
        ],
        out_specs=pl.BlockSpec((_BM, ncls), lambda mi, ni: (mi, 0)),
        out_shape=jax.ShapeDtypeStruct((m, ncls), jnp.float32),
        compiler_params=pltpu.CompilerParams(
            dimension_semantics=("arbitrary", "arbitrary")),
    )(h2, wc1, bc1, wc2, bc2)


def _pad2(x, rows, cols):
    return jnp.pad(x, ((0, rows - x.shape[0]), (0, cols - x.shape[1])))


def kernel(gene_features, train_features, edge_src, edge_dst,
           W_self1, W_neigh1, b1, W_self2, W_neigh2, b2,
           Wc1, bc1, Wc2, bc2):
    f32, bf16 = jnp.float32, jnp.bfloat16

    gp = jnp.concatenate(
        [gene_features,
         jnp.ones((_N_GENE, 1), f32),
         jnp.zeros((_N_GENE, _CPAD - _SRC_DIM - 1), f32)], axis=1)
    gflat = gp.reshape(_ZROW, 128)

    esrc = jnp.pad(edge_src, (0, _EPAD - _E)).reshape(_EROWS, _BE)
    edst = jnp.pad(edge_dst, (0, _EPAD - _E),
                   constant_values=_N_TRAIN).reshape(_EROWS, _BE)

    tfp = _pad2(train_features, _MPAD, _KTRAIN).astype(bf16)
    w1s = _pad2(W_self1, _KTRAIN, _CPAD).astype(bf16)
    w1n = _pad2(W_neigh1, _CPAD, _CPAD).astype(bf16)
    w2s = _pad2(W_self2, _CPAD, _CPAD).astype(bf16)
    w2n = _pad2(W_neigh2, _CPAD, _CPAD).astype(bf16)
    wc1 = _pad2(Wc1, _CPAD, _CPAD).astype(bf16)
    wc2 = _pad2(Wc2, _CPAD, Wc2.shape[1]).astype(bf16)
    b1p = jnp.pad(b1, (0, _CPAD - b1.shape[0])).reshape(1, _CPAD)
    b2p = jnp.pad(b2, (0, _CPAD - b2.shape[0])).reshape(1, _CPAD)
    bc1p = jnp.pad(bc1, (0, _CPAD - bc1.shape[0])).reshape(1, _CPAD)
    bc2p = bc2.reshape(1, -1)

    aggraw = _sc_segment_sum(gflat, esrc, edst)
    s1 = _mm_bias(tfp, w1s, b1p)
    aggb = aggraw.astype(bf16)
    h = _combine1(s1, aggb, w1n)
    h2 = _layer2(h, aggb, w2s, w2n, b2p)
    out = _head(h2, wc1, bc1p, wc2, bc2p)
    return out[:_N_TRAIN]


# speedup vs baseline: 1.5746x; 1.0002x over previous
import functools

import jax
import jax.numpy as jnp
from jax import lax
from jax.experimental import pallas as pl
from jax.experimental.pallas import tpu as pltpu
from jax.experimental.pallas import tpu_sc as plsc

_N_GENE = 2500
_N_TRAIN = 10000
_E = 32000
_SRC_DIM = 2500
_DST_DIM = 2675

_CPAD = 2560
_NCHUNK = _CPAD // 128
_KTRAIN = 2688
_DEG_COL = 2500

_EPAD = 32768
_BE = 128
_NB = _EPAD // (16 * _BE)
_EROWS = _EPAD // _BE
_ZROW = _NCHUNK * _N_GENE
_GROWS = _ZROW + 16
_MPAD = 10240
_STRIPE = _MPAD // 16

_BM = 512
_BN = 1280


def _sc_segment_sum(gflat, esrc, edst):
    mesh = plsc.VectorSubcoreMesh(core_axis_name="c", subcore_axis_name="s")

    @functools.partial(
        pl.kernel,
        mesh=mesh,
        out_type=jax.ShapeDtypeStruct((_MPAD, _CPAD), jnp.float32),
        scratch_types=[
            pltpu.VMEM((_NB, _BE), jnp.int32),
            pltpu.VMEM((_NB, _BE), jnp.int32),
            pltpu.VMEM((_NB, _BE), jnp.int32),
            pltpu.VMEM((_BE, 128), jnp.float32),
            pltpu.VMEM((_BE, 128), jnp.float32),
            pltpu.VMEM((64, 128), jnp.float32),
            pltpu.VMEM_SHARED((_MPAD, 128), jnp.float32),
            pltpu.SemaphoreType.DMA,
            pltpu.SemaphoreType.DMA,
            pltpu.SemaphoreType.DMA,
            pltpu.SemaphoreType.DMA,
        ],
    )
    def k(gflat_hbm, esrc_hbm, edst_hbm, out_hbm,
          src_v, dst_v, idx_v, rows_a, rows_b, zeros_v, acc_sh,
          sga, sgb, ssa, ssb):
        c = lax.axis_index("c")
        s = lax.axis_index("s")

        pltpu.sync_copy(esrc_hbm.at[pl.ds(s * _NB, _NB)], src_v)
        pltpu.sync_copy(edst_hbm.at[pl.ds(s * _NB, _NB)], dst_v)

        zero16 = jnp.zeros((16,), jnp.float32)

        @pl.loop(0, 64)
        def _(i):
            @pl.loop(0, 128, step=16)
            def _(j):
                zeros_v[i, pl.ds(j, 16)] = zero16

        @pl.loop(0, _STRIPE // 64)
        def _(z):
            pltpu.sync_copy(zeros_v, acc_sh.at[pl.ds(s * _STRIPE + z * 64, 64)])

        plsc.subcore_barrier()

        def gather(b, buf, sem):
            pltpu.async_copy(gflat_hbm.at[idx_v.at[b]], buf, sem)

        def wait_gather(b, buf, sem):
            pltpu.make_async_copy(gflat_hbm.at[idx_v.at[b]], buf, sem).wait()

        def scatter(b, buf, sem):
            pltpu.async_copy(buf, acc_sh.at[dst_v.at[b]], sem, add=True)

        def wait_scatter(b, buf, sem):
            pltpu.make_async_copy(buf, acc_sh.at[dst_v.at[b]], sem).wait()

        @pl.loop(0, _NCHUNK // 2)
        def _(cc):
            chunk = c * (_NCHUNK // 2) + cc

            @pl.loop(0, _NB)
            def _(i):
                @pl.loop(0, 128, step=16)
                def _(j):
                    idx_v[i, pl.ds(j, 16)] = (
                        src_v[i, pl.ds(j, 16)] * _NCHUNK + chunk)

            gather(0, rows_a, sga)

            @pl.loop(0, _NB // 2)
            def _(p):
                b0 = p * 2
                wait_gather(b0, rows_a, sga)

                @pl.when(p > 0)
                def _():
                    wait_scatter(b0 - 1, rows_b, ssb)

                gather(b0 + 1, rows_b, sgb)
                scatter(b0, rows_a, ssa)
                wait_gather(b0 + 1, rows_b, sgb)
                wait_scatter(b0, rows_a, ssa)

                @pl.when(p < _NB // 2 - 1)
                def _():
                    gather(b0 + 2, rows_a, sga)

                scatter(b0 + 1, rows_b, ssb)

            wait_scatter(_NB - 1, rows_b, ssb)
            plsc.subcore_barrier()

            pltpu.sync_copy(
                acc_sh.at[pl.ds(s * _STRIPE, _STRIPE)],
                out_hbm.at[pl.ds(s * _STRIPE, _STRIPE),
                           pl.ds(chunk * 128, 128)])

            @pl.loop(0, _STRIPE // 64)
            def _(z):
                pltpu.async_copy(
                    zeros_v, acc_sh.at[pl.ds(s * _STRIPE + z * 64, 64)], sga)

            @pl.loop(0, _STRIPE // 64)
            def _(z):
                pltpu.make_async_copy(
                    zeros_v, acc_sh.at[pl.ds(s * _STRIPE + z * 64, 64)],
                    sga).wait()

            plsc.subcore_barrier()

    return k(gflat, esrc, edst)


def _pad_genes_body(g_ref, o_ref):
    blk = jnp.pad(g_ref[...], ((0, 0), (0, 0), (0, _CPAD - _SRC_DIM)))
    cols = lax.broadcasted_iota(jnp.int32, blk.shape, 2)
    o_ref[...] = jnp.where(cols == _DEG_COL, 1.0, blk)


def _pad_genes(g):
    g4 = g.reshape(4, _N_GENE // 4, _SRC_DIM)
    out = pl.pallas_call(
        _pad_genes_body,
        grid=(4,),
        in_specs=[pl.BlockSpec((1, _N_GENE // 4, _SRC_DIM),
                               lambda mi: (mi, 0, 0))],
        out_specs=pl.BlockSpec((1, _N_GENE // 4, _CPAD),
                               lambda mi: (mi, 0, 0)),
        out_shape=jax.ShapeDtypeStruct((4, _N_GENE // 4, _CPAD), jnp.float32),
    )(g4)
    return out.reshape(_N_GENE, _CPAD)


def _mm_bias_body(x_ref, w_ref, b_ref, o_ref):
    acc = jnp.dot(x_ref[...], w_ref[...], preferred_element_type=jnp.float32)
    acc += b_ref[...].astype(jnp.float32)
    o_ref[...] = acc.astype(jnp.bfloat16)


def _mm_bias(x, w, b):
    m, kdim = x.shape
    n = w.shape[1]
    grid = (n // _BN, m // _BM)
    return pl.pallas_call(
        _mm_bias_body,
        grid=grid,
        in_specs=[
            pl.BlockSpec((_BM, kdim), lambda ni, mi: (mi, 0)),
            pl.BlockSpec((kdim, _BN), lambda ni, mi: (0, ni)),
            pl.BlockSpec((1, _BN), lambda ni, mi: (0, ni)),
        ],
        out_specs=pl.BlockSpec((_BM, _BN), lambda ni, mi: (mi, ni)),
        out_shape=jax.ShapeDtypeStruct((m, n), jnp.bfloat16),
        compiler_params=pltpu.CompilerParams(
            dimension_semantics=("arbitrary", "arbitrary")),
    )(x, w, b)


def _scaled_agg(ag_ref):
    deg = ag_ref[:, _DEG_COL].astype(jnp.float32)
    r = 1.0 / jnp.maximum(deg, 1.0)
    a = ag_ref[...].astype(jnp.float32) * r[:, None]
    return a.astype(jnp.bfloat16)


def _combine1_body(s_ref, ag_ref, wn_ref, o_ref):
    acc = s_ref[...].astype(jnp.float32)
    acc += jnp.dot(_scaled_agg(ag_ref), wn_ref[...],
                   preferred_element_type=jnp.float32)
    o_ref[...] = jnp.maximum(acc, 0.0).astype(jnp.bfloat16)


def _combine1(s1, agg, wn):
    m, n = s1.shape
    grid = (n // _BN, m // _BM)
    return pl.pallas_call(
        _combine1_body,
        grid=grid,
        in_specs=[
            pl.BlockSpec((_BM, _BN), lambda ni, mi: (mi, ni)),
            pl.BlockSpec((_BM, _CPAD), lambda ni, mi: (mi, 0)),
            pl.BlockSpec((_CPAD, _BN), lambda ni, mi: (0, ni)),
        ],
        out_specs=pl.BlockSpec((_BM, _BN), lambda ni, mi: (mi, ni)),
        out_shape=jax.ShapeDtypeStruct((m, n), jnp.bfloat16),
        compiler_params=pltpu.CompilerParams(
            dimension_semantics=("arbitrary", "arbitrary")),
    )(s1, agg, wn)


def _layer2_body(h_ref, ag_ref, ws_ref, wn_ref, b_ref, o_ref):
    acc = jnp.dot(h_ref[...], ws_ref[...], preferred_element_type=jnp.float32)
    acc += jnp.dot(_scaled_agg(ag_ref), wn_ref[...],
                   preferred_element_type=jnp.float32)
    acc += b_ref[...].astype(jnp.float32)
    o_ref[...] = jnp.maximum(acc, 0.0).astype(jnp.bfloat16)


def _layer2(h, agg, ws, wn, b):
    m, kdim = h.shape
    n = ws.shape[1]
    grid = (n // _BN, m // _BM)
    return pl.pallas_call(
        _layer2_body,
        grid=grid,
        in_specs=[
            pl.BlockSpec((_BM, kdim), lambda ni, mi: (mi, 0)),
            pl.BlockSpec((_BM, _CPAD), lambda ni, mi: (mi, 0)),
            pl.BlockSpec((kdim, _BN), lambda ni, mi: (0, ni)),
            pl.BlockSpec((_CPAD, _BN), lambda ni, mi: (0, ni)),
            pl.BlockSpec((1, _BN), lambda ni, mi: (0, ni)),
        ],
        out_specs=pl.BlockSpec((_BM, _BN), lambda ni, mi: (mi, ni)),
        out_shape=jax.ShapeDtypeStruct((m, n), jnp.bfloat16),
        compiler_params=pltpu.CompilerParams(
            dimension_semantics=("arbitrary", "arbitrary")),
    )(h, agg, ws, wn, b)


def _head_body(h2_ref, wc1_ref, bc1_ref, wc2_ref, bc2_ref, o_ref):
    ni = pl.program_id(1)
    t = jnp.dot(h2_ref[...], wc1_ref[...], preferred_element_type=jnp.float32)
    t = jnp.maximum(t + bc1_ref[...].astype(jnp.float32), 0.0)
    part = jnp.dot(t.astype(jnp.bfloat16), wc2_ref[...],
                   preferred_element_type=jnp.float32)

    @pl.when(ni == 0)
    def _():
        o_ref[...] = part + bc2_ref[...].astype(jnp.float32)

    @pl.when(ni > 0)
    def _():
        o_ref[...] += part


def _head(h2, wc1, bc1, wc2, bc2):
    m, kdim = h2.shape
    ncls = wc2.shape[1]
    grid = (m // _BM, kdim // _BN)
    return pl.pallas_call(
        _head_body,
        grid=grid,
        in_specs=[
            pl.BlockSpec((_BM, kdim), lambda mi, ni: (mi, 0)),
            pl.BlockSpec((kdim, _BN), lambda mi, ni: (0, ni)),
            pl.BlockSpec((1, _BN), lambda mi, ni: (0, ni)),
            pl.BlockSpec((_BN, ncls), lambda mi, ni: (ni, 0)),
            pl.BlockSpec((1, ncls), lambda mi, ni: (0, 0)),
        ],
        out_specs=pl.BlockSpec((_BM, ncls), lambda mi, ni: (mi, 0)),
        out_shape=jax.ShapeDtypeStruct((m, ncls), jnp.float32),
        compiler_params=pltpu.CompilerParams(
            dimension_semantics=("arbitrary", "arbitrary")),
    )(h2, wc1, bc1, wc2, bc2)


def _pad2(x, rows, cols):
    return jnp.pad(x, ((0, rows - x.shape[0]), (0, cols - x.shape[1])))


def kernel(gene_features, train_features, edge_src, edge_dst,
           W_self1, W_neigh1, b1, W_self2, W_neigh2, b2,
           Wc1, bc1, Wc2, bc2):
    f32, bf16 = jnp.float32, jnp.bfloat16

    gflat = _pad_genes(gene_features).reshape(_ZROW, 128)

    esrc = jnp.pad(edge_src, (0, _EPAD - _E)).reshape(_EROWS, _BE)
    edst = jnp.pad(edge_dst, (0, _EPAD - _E),
                   constant_values=_N_TRAIN).reshape(_EROWS, _BE)

    tfp = _pad2(train_features, _MPAD, _KTRAIN).astype(bf16)
    w1s = _pad2(W_self1, _KTRAIN, _CPAD).astype(bf16)
    w1n = _pad2(W_neigh1, _CPAD, _CPAD).astype(bf16)
    w2s = _pad2(W_self2, _CPAD, _CPAD).astype(bf16)
    w2n = _pad2(W_neigh2, _CPAD, _CPAD).astype(bf16)
    wc1 = _pad2(Wc1, _CPAD, _CPAD).astype(bf16)
    wc2 = _pad2(Wc2, _CPAD, Wc2.shape[1]).astype(bf16)
    b1p = jnp.pad(b1, (0, _CPAD - b1.shape[0])).reshape(1, _CPAD)
    b2p = jnp.pad(b2, (0, _CPAD - b2.shape[0])).reshape(1, _CPAD)
    bc1p = jnp.pad(bc1, (0, _CPAD - bc1.shape[0])).reshape(1, _CPAD)
    bc2p = bc2.reshape(1, -1)

    aggraw = _sc_segment_sum(gflat, esrc, edst)
    s1 = _mm_bias(tfp, w1s, b1p)
    aggb = aggraw.astype(bf16)
    h = _combine1(s1, aggb, w1n)
    h2 = _layer2(h, aggb, w2s, w2n, b2p)
    out = _head(h2, wc1, bc1p, wc2, bc2p)
    return out[:_N_TRAIN]


# speedup vs baseline: 1.6824x; 1.0685x over previous
import functools

import jax
import jax.numpy as jnp
from jax import lax
from jax.experimental import pallas as pl
from jax.experimental.pallas import tpu as pltpu
from jax.experimental.pallas import tpu_sc as plsc

_N_GENE = 2500
_N_TRAIN = 10000
_E = 32000
_SRC_DIM = 2500
_DST_DIM = 2675

_CPAD = 2560
_NCHUNK = _CPAD // 128
_KTRAIN = 2688
_DEG_COL = 2500

_EPAD = 32768
_BE = 128
_NB = _EPAD // (16 * _BE)
_EROWS = _EPAD // _BE
_ZROW = _NCHUNK * _N_GENE
_GROWS = _ZROW + 16
_MPAD = 10240
_STRIPE = _MPAD // 16

_BM = 512
_BN = 1280


def _sc_segment_sum(gflat, esrc, edst, base):
    mesh = plsc.VectorSubcoreMesh(core_axis_name="c", subcore_axis_name="s")
    ncol = (_NCHUNK // 2) * 128

    @functools.partial(
        pl.kernel,
        mesh=mesh,
        out_type=jax.ShapeDtypeStruct((_MPAD, ncol), jnp.float32),
        scratch_types=[
            pltpu.VMEM((_NB, _BE), jnp.int32),
            pltpu.VMEM((_NB, _BE), jnp.int32),
            pltpu.VMEM((_NB, _BE), jnp.int32),
            pltpu.VMEM((_BE, 128), jnp.float32),
            pltpu.VMEM((_BE, 128), jnp.float32),
            pltpu.VMEM((64, 128), jnp.float32),
            pltpu.VMEM_SHARED((_MPAD, 128), jnp.float32),
            pltpu.SemaphoreType.DMA,
            pltpu.SemaphoreType.DMA,
            pltpu.SemaphoreType.DMA,
            pltpu.SemaphoreType.DMA,
        ],
    )
    def k(gflat_hbm, esrc_hbm, edst_hbm, out_hbm,
          src_v, dst_v, idx_v, rows_a, rows_b, zeros_v, acc_sh,
          sga, sgb, ssa, ssb):
        c = lax.axis_index("c")
        s = lax.axis_index("s")

        pltpu.sync_copy(esrc_hbm.at[pl.ds(s * _NB, _NB)], src_v)
        pltpu.sync_copy(edst_hbm.at[pl.ds(s * _NB, _NB)], dst_v)

        zero16 = jnp.zeros((16,), jnp.float32)

        @pl.loop(0, 64)
        def _(i):
            @pl.loop(0, 128, step=16)
            def _(j):
                zeros_v[i, pl.ds(j, 16)] = zero16

        @pl.loop(0, _STRIPE // 64)
        def _(z):
            pltpu.sync_copy(zeros_v, acc_sh.at[pl.ds(s * _STRIPE + z * 64, 64)])

        plsc.subcore_barrier()

        def gather(b, buf, sem):
            pltpu.async_copy(gflat_hbm.at[idx_v.at[b]], buf, sem)

        def wait_gather(b, buf, sem):
            pltpu.make_async_copy(gflat_hbm.at[idx_v.at[b]], buf, sem).wait()

        def scatter(b, buf, sem):
            pltpu.async_copy(buf, acc_sh.at[dst_v.at[b]], sem, add=True)

        def wait_scatter(b, buf, sem):
            pltpu.make_async_copy(buf, acc_sh.at[dst_v.at[b]], sem).wait()

        @pl.loop(0, _NCHUNK // 4)
        def _(cc):
            local = c * (_NCHUNK // 4) + cc
            chunk = base + local

            @pl.loop(0, _NB)
            def _(i):
                @pl.loop(0, 128, step=16)
                def _(j):
                    idx_v[i, pl.ds(j, 16)] = (
                        src_v[i, pl.ds(j, 16)] * _NCHUNK + chunk)

            gather(0, rows_a, sga)

            @pl.loop(0, _NB // 2)
            def _(p):
                b0 = p * 2
                wait_gather(b0, rows_a, sga)

                @pl.when(p > 0)
                def _():
                    wait_scatter(b0 - 1, rows_b, ssb)

                gather(b0 + 1, rows_b, sgb)
                scatter(b0, rows_a, ssa)
                wait_gather(b0 + 1, rows_b, sgb)
                wait_scatter(b0, rows_a, ssa)

                @pl.when(p < _NB // 2 - 1)
                def _():
                    gather(b0 + 2, rows_a, sga)

                scatter(b0 + 1, rows_b, ssb)

            wait_scatter(_NB - 1, rows_b, ssb)
            plsc.subcore_barrier()

            pltpu.sync_copy(
                acc_sh.at[pl.ds(s * _STRIPE, _STRIPE)],
                out_hbm.at[pl.ds(s * _STRIPE, _STRIPE),
                           pl.ds(local * 128, 128)])

            @pl.loop(0, _STRIPE // 64)
            def _(z):
                pltpu.async_copy(
                    zeros_v, acc_sh.at[pl.ds(s * _STRIPE + z * 64, 64)], sga)

            @pl.loop(0, _STRIPE // 64)
            def _(z):
                pltpu.make_async_copy(
                    zeros_v, acc_sh.at[pl.ds(s * _STRIPE + z * 64, 64)],
                    sga).wait()

            plsc.subcore_barrier()

    return k(gflat, esrc, edst)


def _pad_genes_body(g_ref, o_ref):
    blk = jnp.pad(g_ref[...], ((0, 0), (0, 0), (0, _CPAD - _SRC_DIM)))
    cols = lax.broadcasted_iota(jnp.int32, blk.shape, 2)
    o_ref[...] = jnp.where(cols == _DEG_COL, 1.0, blk)


def _pad_genes(g):
    g4 = g.reshape(4, _N_GENE // 4, _SRC_DIM)
    out = pl.pallas_call(
        _pad_genes_body,
        grid=(4,),
        in_specs=[pl.BlockSpec((1, _N_GENE // 4, _SRC_DIM),
                               lambda mi: (mi, 0, 0))],
        out_specs=pl.BlockSpec((1, _N_GENE // 4, _CPAD),
                               lambda mi: (mi, 0, 0)),
        out_shape=jax.ShapeDtypeStruct((4, _N_GENE // 4, _CPAD), jnp.float32),
    )(g4)
    return out.reshape(_N_GENE, _CPAD)


def _mm_bias_body(x_ref, w_ref, b_ref, o_ref):
    acc = jnp.dot(x_ref[...], w_ref[...], preferred_element_type=jnp.float32)
    acc += b_ref[...].astype(jnp.float32)
    o_ref[...] = acc.astype(jnp.bfloat16)


def _mm_bias(x, w, b):
    m, kdim = x.shape
    n = w.shape[1]
    grid = (n // _BN, m // _BM)
    return pl.pallas_call(
        _mm_bias_body,
        grid=grid,
        in_specs=[
            pl.BlockSpec((_BM, kdim), lambda ni, mi: (mi, 0)),
            pl.BlockSpec((kdim, _BN), lambda ni, mi: (0, ni)),
            pl.BlockSpec((1, _BN), lambda ni, mi: (0, ni)),
        ],
        out_specs=pl.BlockSpec((_BM, _BN), lambda ni, mi: (mi, ni)),
        out_shape=jax.ShapeDtypeStruct((m, n), jnp.bfloat16),
        compiler_params=pltpu.CompilerParams(
            dimension_semantics=("arbitrary", "arbitrary")),
    )(x, w, b)


_HALF = (_NCHUNK // 2) * 128
_DEG_LOCAL = _DEG_COL - _HALF


def _dual_mm_body(a_ref, w1_ref, w2_ref, o1_ref, o2_ref):
    a = a_ref[...].astype(jnp.bfloat16)
    o1_ref[...] = jnp.dot(a, w1_ref[...],
                          preferred_element_type=jnp.float32).astype(jnp.bfloat16)
    o2_ref[...] = jnp.dot(a, w2_ref[...],
                          preferred_element_type=jnp.float32).astype(jnp.bfloat16)


def _dual_mm(aggl, w1, w2):
    m = aggl.shape[0]
    n = w1.shape[1]
    grid = (n // _BN, m // _BM)
    spec_out = pl.BlockSpec((_BM, _BN), lambda ni, mi: (mi, ni))
    return pl.pallas_call(
        _dual_mm_body,
        grid=grid,
        in_specs=[
            pl.BlockSpec((_BM, _HALF), lambda ni, mi: (mi, 0)),
            pl.BlockSpec((_HALF, _BN), lambda ni, mi: (0, ni)),
            pl.BlockSpec((_HALF, _BN), lambda ni, mi: (0, ni)),
        ],
        out_specs=[spec_out, spec_out],
        out_shape=[jax.ShapeDtypeStruct((m, n), jnp.bfloat16)] * 2,
        compiler_params=pltpu.CompilerParams(
            dimension_semantics=("arbitrary", "arbitrary")),
    )(aggl, w1, w2)


def _recip_deg(agr_ref):
    deg = agr_ref[:, _DEG_LOCAL]
    return 1.0 / jnp.maximum(deg, 1.0)


def _combine_a_body(s_ref, q_ref, agr_ref, wn_ref, o_ref):
    q = q_ref[...].astype(jnp.float32)
    q += jnp.dot(agr_ref[...].astype(jnp.bfloat16), wn_ref[...],
                 preferred_element_type=jnp.float32)
    acc = s_ref[...].astype(jnp.float32) + _recip_deg(agr_ref)[:, None] * q
    o_ref[...] = jnp.maximum(acc, 0.0).astype(jnp.bfloat16)


def _combine_a(s1, q1, aggr, wn):
    m, n = s1.shape
    grid = (n // _BN, m // _BM)
    return pl.pallas_call(
        _combine_a_body,
        grid=grid,
        in_specs=[
            pl.BlockSpec((_BM, _BN), lambda ni, mi: (mi, ni)),
            pl.BlockSpec((_BM, _BN), lambda ni, mi: (mi, ni)),
            pl.BlockSpec((_BM, _HALF), lambda ni, mi: (mi, 0)),
            pl.BlockSpec((_HALF, _BN), lambda ni, mi: (0, ni)),
        ],
        out_specs=pl.BlockSpec((_BM, _BN), lambda ni, mi: (mi, ni)),
        out_shape=jax.ShapeDtypeStruct((m, n), jnp.bfloat16),
        compiler_params=pltpu.CompilerParams(
            dimension_semantics=("arbitrary", "arbitrary")),
    )(s1, q1, aggr, wn)


def _combine_b_body(h_ref, ws_ref, q_ref, agr_ref, wn_ref, b_ref, o_ref):
    q = q_ref[...].astype(jnp.float32)
    q += jnp.dot(agr_ref[...].astype(jnp.bfloat16), wn_ref[...],
                 preferred_element_type=jnp.float32)
    acc = jnp.dot(h_ref[...], ws_ref[...], preferred_element_type=jnp.float32)
    acc += _recip_deg(agr_ref)[:, None] * q
    acc += b_ref[...].astype(jnp.float32)
    o_ref[...] = jnp.maximum(acc, 0.0).astype(jnp.bfloat16)


def _combine_b(h, ws, q2, aggr, wn, b):
    m, kdim = h.shape
    n = ws.shape[1]
    grid = (n // _BN, m // _BM)
    return pl.pallas_call(
        _combine_b_body,
        grid=grid,
        in_specs=[
            pl.BlockSpec((_BM, kdim), lambda ni, mi: (mi, 0)),
            pl.BlockSpec((kdim, _BN), lambda ni, mi: (0, ni)),
            pl.BlockSpec((_BM, _BN), lambda ni, mi: (mi, ni)),
            pl.BlockSpec((_BM, _HALF), lambda ni, mi: (mi, 0)),
            pl.BlockSpec((_HALF, _BN), lambda ni, mi: (0, ni)),
            pl.BlockSpec((1, _BN), lambda ni, mi: (0, ni)),
        ],
        out_specs=pl.BlockSpec((_BM, _BN), lambda ni, mi: (mi, ni)),
        out_shape=jax.ShapeDtypeStruct((m, n), jnp.bfloat16),
        compiler_params=pltpu.CompilerParams(
            dimension_semantics=("arbitrary", "arbitrary")),
    )(h, ws, q2, aggr, wn, b)


def _head_body(h2_ref, wc1_ref, bc1_ref, wc2_ref, bc2_ref, o_ref):
    ni = pl.program_id(1)
    t = jnp.dot(h2_ref[...], wc1_ref[...], preferred_element_type=jnp.float32)
    t = jnp.maximum(t + bc1_ref[...].astype(jnp.float32), 0.0)
    part = jnp.dot(t.astype(jnp.bfloat16), wc2_ref[...],
                   preferred_element_type=jnp.float32)

    @pl.when(ni == 0)
    def _():
        o_ref[...] = part + bc2_ref[...].astype(jnp.float32)

    @pl.when(ni > 0)
    def _():
        o_ref[...] += part


def _head(h2, wc1, bc1, wc2, bc2):
    m, kdim = h2.shape
    ncls = wc2.shape[1]
    grid = (m // _BM, kdim // _BN)
    return pl.pallas_call(
        _head_body,
        grid=grid,
        in_specs=[
            pl.BlockSpec((_BM, kdim), lambda mi, ni: (mi, 0)),
            pl.BlockSpec((kdim, _BN), lambda mi, ni: (0, ni)),
            pl.BlockSpec((1, _BN), lambda mi, ni: (0, ni)),
            pl.BlockSpec((_BN, ncls), lambda mi, ni: (ni, 0)),
            pl.BlockSpec((1, ncls), lambda mi, ni: (0, 0)),
        ],
        out_specs=pl.BlockSpec((_BM, ncls), lambda mi, ni: (mi, 0)),
        out_shape=jax.ShapeDtypeStruct((m, ncls), jnp.float32),
        compiler_params=pltpu.CompilerParams(
            dimension_semantics=("arbitrary", "arbitrary")),
    )(h2, wc1, bc1, wc2, bc2)


def _pad2(x, rows, cols):
    return jnp.pad(x, ((0, rows - x.shape[0]), (0, cols - x.shape[1])))


def kernel(gene_features, train_features, edge_src, edge_dst,
           W_self1, W_neigh1, b1, W_self2, W_neigh2, b2,
           Wc1, bc1, Wc2, bc2):
    f32, bf16 = jnp.float32, jnp.bfloat16

    gflat = _pad_genes(gene_features).reshape(_ZROW, 128)

    esrc = jnp.pad(edge_src, (0, _EPAD - _E)).reshape(_EROWS, _BE)
    edst = jnp.pad(edge_dst, (0, _EPAD - _E),
                   constant_values=_N_TRAIN).reshape(_EROWS, _BE)

    tfp = _pad2(train_features, _MPAD, _KTRAIN).astype(bf16)
    w1s = _pad2(W_self1, _KTRAIN, _CPAD).astype(bf16)
    w1nl = _pad2(W_neigh1[:_HALF], _HALF, _CPAD).astype(bf16)
    w1nr = _pad2(W_neigh1[_HALF:], _HALF, _CPAD).astype(bf16)
    w2nl = _pad2(W_neigh2[:_HALF], _HALF, _CPAD).astype(bf16)
    w2nr = _pad2(W_neigh2[_HALF:], _HALF, _CPAD).astype(bf16)
    w2s = _pad2(W_self2, _CPAD, _CPAD).astype(bf16)
    wc1 = _pad2(Wc1, _CPAD, _CPAD).astype(bf16)
    wc2 = _pad2(Wc2, _CPAD, Wc2.shape[1]).astype(bf16)
    b1p = jnp.pad(b1, (0, _CPAD - b1.shape[0])).reshape(1, _CPAD)
    b2p = jnp.pad(b2, (0, _CPAD - b2.shape[0])).reshape(1, _CPAD)
    bc1p = jnp.pad(bc1, (0, _CPAD - bc1.shape[0])).reshape(1, _CPAD)
    bc2p = bc2.reshape(1, -1)

    aggl = _sc_segment_sum(gflat, esrc, edst, 0)
    aggr = _sc_segment_sum(gflat, esrc, edst, _NCHUNK // 2)
    s1 = _mm_bias(tfp, w1s, b1p)
    q1, q2 = _dual_mm(aggl, w1nl, w2nl)
    h = _combine_a(s1, q1, aggr, w1nr)
    h2 = _combine_b(h, w2s, q2, aggr, w2nr, b2p)
    out = _head(h2, wc1, bc1p, wc2, bc2p)
    return out[:_N_TRAIN]


# speedup vs baseline: 1.8387x; 1.0929x over previous
import functools

import jax
import jax.numpy as jnp
from jax import lax
from jax.experimental import pallas as pl
from jax.experimental.pallas import tpu as pltpu
from jax.experimental.pallas import tpu_sc as plsc

_N_GENE = 2500
_N_TRAIN = 10000
_E = 32000
_SRC_DIM = 2500
_DST_DIM = 2675

_CPAD = 2560
_NCHUNK = _CPAD // 128
_KTRAIN = 2688
_DEG_COL = 2500

_EPAD = 32768
_BE = 128
_NB = _EPAD // (16 * _BE)
_EROWS = _EPAD // _BE
_ZROW = _NCHUNK * _N_GENE
_GROWS = _ZROW + 16
_MPAD = 10240
_STRIPE = _MPAD // 16

_BM = 400
_BN = 1280


def _sc_segment_sum(gflat, esrc, edst, base):
    mesh = plsc.VectorSubcoreMesh(core_axis_name="c", subcore_axis_name="s")
    ncol = (_NCHUNK // 2) * 128

    @functools.partial(
        pl.kernel,
        mesh=mesh,
        out_type=jax.ShapeDtypeStruct((_MPAD, ncol), jnp.float32),
        scratch_types=[
            pltpu.VMEM((_NB, _BE), jnp.int32),
            pltpu.VMEM((_NB, _BE), jnp.int32),
            pltpu.VMEM((_NB, _BE), jnp.int32),
            pltpu.VMEM((_BE, 128), jnp.float32),
            pltpu.VMEM((_BE, 128), jnp.float32),
            pltpu.VMEM((64, 128), jnp.float32),
            pltpu.VMEM_SHARED((_MPAD, 128), jnp.float32),
            pltpu.SemaphoreType.DMA,
            pltpu.SemaphoreType.DMA,
            pltpu.SemaphoreType.DMA,
            pltpu.SemaphoreType.DMA,
        ],
    )
    def k(gflat_hbm, esrc_hbm, edst_hbm, out_hbm,
          src_v, dst_v, idx_v, rows_a, rows_b, zeros_v, acc_sh,
          sga, sgb, ssa, ssb):
        c = lax.axis_index("c")
        s = lax.axis_index("s")

        pltpu.sync_copy(esrc_hbm.at[pl.ds(s * _NB, _NB)], src_v)
        pltpu.sync_copy(edst_hbm.at[pl.ds(s * _NB, _NB)], dst_v)

        zero16 = jnp.zeros((16,), jnp.float32)

        @pl.loop(0, 64)
        def _(i):
            @pl.loop(0, 128, step=16)
            def _(j):
                zeros_v[i, pl.ds(j, 16)] = zero16

        @pl.loop(0, _STRIPE // 64)
        def _(z):
            pltpu.sync_copy(zeros_v, acc_sh.at[pl.ds(s * _STRIPE + z * 64, 64)])

        plsc.subcore_barrier()

        def gather(b, buf, sem):
            pltpu.async_copy(gflat_hbm.at[idx_v.at[b]], buf, sem)

        def wait_gather(b, buf, sem):
            pltpu.make_async_copy(gflat_hbm.at[idx_v.at[b]], buf, sem).wait()

        def scatter(b, buf, sem):
            pltpu.async_copy(buf, acc_sh.at[dst_v.at[b]], sem, add=True)

        def wait_scatter(b, buf, sem):
            pltpu.make_async_copy(buf, acc_sh.at[dst_v.at[b]], sem).wait()

        @pl.loop(0, _NCHUNK // 4)
        def _(cc):
            local = c * (_NCHUNK // 4) + cc
            chunk = base + local

            @pl.loop(0, _NB)
            def _(i):
                @pl.loop(0, 128, step=16)
                def _(j):
                    idx_v[i, pl.ds(j, 16)] = (
                        src_v[i, pl.ds(j, 16)] * _NCHUNK + chunk)

            gather(0, rows_a, sga)

            @pl.loop(0, _NB // 2)
            def _(p):
                b0 = p * 2
                wait_gather(b0, rows_a, sga)

                @pl.when(p > 0)
                def _():
                    wait_scatter(b0 - 1, rows_b, ssb)

                gather(b0 + 1, rows_b, sgb)
                scatter(b0, rows_a, ssa)
                wait_gather(b0 + 1, rows_b, sgb)
                wait_scatter(b0, rows_a, ssa)

                @pl.when(p < _NB // 2 - 1)
                def _():
                    gather(b0 + 2, rows_a, sga)

                scatter(b0 + 1, rows_b, ssb)

            wait_scatter(_NB - 1, rows_b, ssb)
            plsc.subcore_barrier()

            pltpu.sync_copy(
                acc_sh.at[pl.ds(s * _STRIPE, _STRIPE)],
                out_hbm.at[pl.ds(s * _STRIPE, _STRIPE),
                           pl.ds(local * 128, 128)])

            @pl.loop(0, _STRIPE // 64)
            def _(z):
                pltpu.async_copy(
                    zeros_v, acc_sh.at[pl.ds(s * _STRIPE + z * 64, 64)], sga)

            @pl.loop(0, _STRIPE // 64)
            def _(z):
                pltpu.make_async_copy(
                    zeros_v, acc_sh.at[pl.ds(s * _STRIPE + z * 64, 64)],
                    sga).wait()

            plsc.subcore_barrier()

    return k(gflat, esrc, edst)


def _pad_genes_body(g_ref, o_ref):
    blk = jnp.pad(g_ref[...], ((0, 0), (0, 0), (0, _CPAD - _SRC_DIM)))
    cols = lax.broadcasted_iota(jnp.int32, blk.shape, 2)
    o_ref[...] = jnp.where(cols == _DEG_COL, 1.0, blk)


def _pad_genes(g):
    g4 = g.reshape(4, _N_GENE // 4, _SRC_DIM)
    out = pl.pallas_call(
        _pad_genes_body,
        grid=(4,),
        in_specs=[pl.BlockSpec((1, _N_GENE // 4, _SRC_DIM),
                               lambda mi: (mi, 0, 0))],
        out_specs=pl.BlockSpec((1, _N_GENE // 4, _CPAD),
                               lambda mi: (mi, 0, 0)),
        out_shape=jax.ShapeDtypeStruct((4, _N_GENE // 4, _CPAD), jnp.float32),
    )(g4)
    return out.reshape(_N_GENE, _CPAD)


def _mm_bias_body(x_ref, w_ref, b_ref, o_ref):
    acc = jnp.dot(x_ref[...].astype(jnp.bfloat16), w_ref[...],
                  preferred_element_type=jnp.float32)
    acc += b_ref[...].astype(jnp.float32)
    o_ref[...] = acc.astype(jnp.bfloat16)


def _mm_bias(x, w, b):
    m, kdim = x.shape
    n = w.shape[1]
    return pl.pallas_call(
        _mm_bias_body,
        grid=(m // _BM,),
        in_specs=[
            pl.BlockSpec((_BM, kdim), lambda mi: (mi, 0)),
            pl.BlockSpec((kdim, n), lambda mi: (0, 0)),
            pl.BlockSpec((1, n), lambda mi: (0, 0)),
        ],
        out_specs=pl.BlockSpec((_BM, n), lambda mi: (mi, 0)),
        out_shape=jax.ShapeDtypeStruct((m, n), jnp.bfloat16),
        compiler_params=pltpu.CompilerParams(
            dimension_semantics=("arbitrary",)),
    )(x, w, b)


_HALF = (_NCHUNK // 2) * 128
_DEG_LOCAL = _DEG_COL - _HALF


def _dual_mm_body(a_ref, w1_ref, w2_ref, o1_ref, o2_ref):
    a = a_ref[...].astype(jnp.bfloat16)
    o1_ref[...] = jnp.dot(a, w1_ref[...],
                          preferred_element_type=jnp.float32).astype(jnp.bfloat16)
    o2_ref[...] = jnp.dot(a, w2_ref[...],
                          preferred_element_type=jnp.float32).astype(jnp.bfloat16)


def _dual_mm(aggl, w1, w2, m):
    n = w1.shape[1]
    spec_out = pl.BlockSpec((_BM, n), lambda mi: (mi, 0))
    return pl.pallas_call(
        _dual_mm_body,
        grid=(m // _BM,),
        in_specs=[
            pl.BlockSpec((_BM, _HALF), lambda mi: (mi, 0)),
            pl.BlockSpec((_HALF, n), lambda mi: (0, 0)),
            pl.BlockSpec((_HALF, n), lambda mi: (0, 0)),
        ],
        out_specs=[spec_out, spec_out],
        out_shape=[jax.ShapeDtypeStruct((m, n), jnp.bfloat16)] * 2,
        compiler_params=pltpu.CompilerParams(
            dimension_semantics=("arbitrary",)),
    )(aggl, w1, w2)


def _recip_deg(agr_ref):
    deg = agr_ref[:, _DEG_LOCAL]
    return 1.0 / jnp.maximum(deg, 1.0)


def _combine_a_body(s_ref, q_ref, agr_ref, wn_ref, o_ref):
    q = q_ref[...].astype(jnp.float32)
    q += jnp.dot(agr_ref[:, :_DEG_LOCAL].astype(jnp.bfloat16), wn_ref[...],
                 preferred_element_type=jnp.float32)
    acc = s_ref[...].astype(jnp.float32) + _recip_deg(agr_ref)[:, None] * q
    o_ref[...] = jnp.maximum(acc, 0.0).astype(jnp.bfloat16)


def _combine_a(s1, q1, aggr, wn):
    m, n = s1.shape
    return pl.pallas_call(
        _combine_a_body,
        grid=(m // _BM,),
        in_specs=[
            pl.BlockSpec((_BM, n), lambda mi: (mi, 0)),
            pl.BlockSpec((_BM, n), lambda mi: (mi, 0)),
            pl.BlockSpec((_BM, _HALF), lambda mi: (mi, 0)),
            pl.BlockSpec((_DEG_LOCAL, n), lambda mi: (0, 0)),
        ],
        out_specs=pl.BlockSpec((_BM, n), lambda mi: (mi, 0)),
        out_shape=jax.ShapeDtypeStruct((m, n), jnp.bfloat16),
        compiler_params=pltpu.CompilerParams(
            dimension_semantics=("arbitrary",)),
    )(s1, q1, aggr, wn)


def _combine_b_body(h_ref, ws_ref, q_ref, agr_ref, wn_ref, b_ref, o_ref):
    q = q_ref[...].astype(jnp.float32)
    q += jnp.dot(agr_ref[:, :_DEG_LOCAL].astype(jnp.bfloat16), wn_ref[...],
                 preferred_element_type=jnp.float32)
    acc = jnp.dot(h_ref[...], ws_ref[...], preferred_element_type=jnp.float32)
    acc += _recip_deg(agr_ref)[:, None] * q
    acc += b_ref[...].astype(jnp.float32)
    o_ref[...] = jnp.maximum(acc, 0.0).astype(jnp.bfloat16)


def _combine_b(h, ws, q2, aggr, wn, b):
    m, kdim = h.shape
    n = ws.shape[1]
    return pl.pallas_call(
        _combine_b_body,
        grid=(m // _BM,),
        in_specs=[
            pl.BlockSpec((_BM, kdim), lambda mi: (mi, 0)),
            pl.BlockSpec((kdim, n), lambda mi: (0, 0)),
            pl.BlockSpec((_BM, n), lambda mi: (mi, 0)),
            pl.BlockSpec((_BM, _HALF), lambda mi: (mi, 0)),
            pl.BlockSpec((_DEG_LOCAL, n), lambda mi: (0, 0)),
            pl.BlockSpec((1, n), lambda mi: (0, 0)),
        ],
        out_specs=pl.BlockSpec((_BM, n), lambda mi: (mi, 0)),
        out_shape=jax.ShapeDtypeStruct((m, n), jnp.bfloat16),
        compiler_params=pltpu.CompilerParams(
            dimension_semantics=("arbitrary",)),
    )(h, ws, q2, aggr, wn, b)


def _head_body(h2_ref, wc1_ref, bc1_ref, wc2_ref, bc2_ref, o_ref):
    t = jnp.dot(h2_ref[...], wc1_ref[...], preferred_element_type=jnp.float32)
    t = jnp.maximum(t + bc1_ref[...].astype(jnp.float32), 0.0)
    out = jnp.dot(t.astype(jnp.bfloat16), wc2_ref[...],
                  preferred_element_type=jnp.float32)
    o_ref[...] = out + bc2_ref[...].astype(jnp.float32)


def _head(h2, wc1, bc1, wc2, bc2):
    m, kdim = h2.shape
    ncls = wc2.shape[1]
    return pl.pallas_call(
        _head_body,
        grid=(m // _BM,),
        in_specs=[
            pl.BlockSpec((_BM, kdim), lambda mi: (mi, 0)),
            pl.BlockSpec((kdim, kdim), lambda mi: (0, 0)),
            pl.BlockSpec((1, kdim), lambda mi: (0, 0)),
            pl.BlockSpec((kdim, ncls), lambda mi: (0, 0)),
            pl.BlockSpec((1, ncls), lambda mi: (0, 0)),
        ],
        out_specs=pl.BlockSpec((_BM, ncls), lambda mi: (mi, 0)),
        out_shape=jax.ShapeDtypeStruct((m, ncls), jnp.float32),
        compiler_params=pltpu.CompilerParams(
            dimension_semantics=("arbitrary",)),
    )(h2, wc1, bc1, wc2, bc2)


def _pad2(x, rows, cols):
    return jnp.pad(x, ((0, rows - x.shape[0]), (0, cols - x.shape[1])))


def kernel(gene_features, train_features, edge_src, edge_dst,
           W_self1, W_neigh1, b1, W_self2, W_neigh2, b2,
           Wc1, bc1, Wc2, bc2):
    bf16 = jnp.bfloat16

    gflat = _pad_genes(gene_features).reshape(_ZROW, 128)

    esrc = jnp.pad(edge_src, (0, _EPAD - _E)).reshape(_EROWS, _BE)
    edst = jnp.pad(edge_dst, (0, _EPAD - _E),
                   constant_values=_N_TRAIN).reshape(_EROWS, _BE)

    w1s = W_self1.astype(bf16)
    w1nl = W_neigh1[:_HALF].astype(bf16)
    w1nr = W_neigh1[_HALF:].astype(bf16)
    w2nl = W_neigh2[:_HALF].astype(bf16)
    w2nr = W_neigh2[_HALF:].astype(bf16)
    w2s = W_self2.astype(bf16)
    wc1 = Wc1.astype(bf16)
    wc2 = Wc2.astype(bf16)
    b1p = b1.reshape(1, -1)
    b2p = b2.reshape(1, -1)
    bc1p = bc1.reshape(1, -1)
    bc2p = bc2.reshape(1, -1)

    aggl = _sc_segment_sum(gflat, esrc, edst, 0)
    aggr = _sc_segment_sum(gflat, esrc, edst, _NCHUNK // 2)
    s1 = _mm_bias(train_features, w1s, b1p)
    q1, q2 = _dual_mm(aggl, w1nl, w2nl, _N_TRAIN)
    h = _combine_a(s1, q1, aggr, w1nr)
    h2 = _combine_b(h, w2s, q2, aggr, w2nr, b2p)
    return _head(h2, wc1, bc1p, wc2, bc2p)
